# baseline (speedup 1.0000x reference)
import functools

import jax
import jax.numpy as jnp
import numpy as np
from jax.experimental import pallas as pl
from jax.experimental.pallas import tpu as pltpu
from jax.experimental.pallas import tpu_sc as plsc

B, S, D = 2, 2048, 768
E, C, F = 64, 64, 3072
BS = B * S
BC = B * C
EBC = E * BC
NROWS = EBC + BS
SQ2PI = 0.7978845608028654

_HI = jax.lax.Precision.HIGHEST


def _router_body(x_ref, wr_ref, br_ref, dsp_ref, cmb_ref, scs_ref, init_ref):
    x = x_ref[:]
    logits = jnp.dot(x.reshape(BS, D), wr_ref[:],
                     preferred_element_type=jnp.float32) + br_ref[:]
    l3 = logits.reshape(B, S, E)
    m3 = jnp.max(l3, axis=-1, keepdims=True)
    ssum = jnp.sum(jnp.exp(l3 - m3), axis=-1, keepdims=True)
    mp3 = 1.0 / ssum
    ie = jax.lax.broadcasted_iota(jnp.int32, (B, S, E), 2)
    idx3 = jnp.min(jnp.where(l3 == m3, ie, E), axis=-1)
    oh = (ie == idx3[:, :, None]).astype(jnp.float32)
    cum = oh
    k = 1
    while k < S:
        cum = cum + jnp.concatenate(
            [jnp.zeros((B, k, E), jnp.float32), cum[:, :S - k, :]], axis=1)
        k *= 2
    posf = jnp.sum(cum * oh, axis=-1) - 1.0
    ic = jax.lax.broadcasted_iota(jnp.int32, (B, S, C), 2).astype(jnp.float32)
    poh = (ic == posf[:, :, None]).astype(jnp.float32)
    s1 = jax.lax.broadcasted_iota(jnp.int32, (B, S, E), 1).astype(
        jnp.float32) + 1.0
    dn = (((0,), (0,)), ((), ()))
    dsp_cols, sc_cols = [], []
    for b in range(B):
        stb = jax.lax.dot_general(oh[b] * s1[b], poh[b], dn, precision=_HI)
        scb = jax.lax.dot_general(oh[b] * mp3[b], poh[b], dn, precision=_HI)
        t = stb.astype(jnp.int32) - 1
        dummy = (jax.lax.broadcasted_iota(jnp.int32, (E, C), 0) * BC
                 + jax.lax.broadcasted_iota(jnp.int32, (E, C), 1)
                 + b * C) % BS
        dsp_cols.append(jnp.where(t >= 0, t + b * S, dummy))
        sc_cols.append(scb)
    dsp_ref[:] = jnp.concatenate(dsp_cols, axis=1)
    scs_ref[:] = jnp.concatenate(sc_cols, axis=1)[:, None, :]
    pos_i = posf.astype(jnp.int32)
    within = posf < float(C)
    bidx = jax.lax.broadcasted_iota(jnp.int32, (B, S), 0)
    sidx = jax.lax.broadcasted_iota(jnp.int32, (B, S), 1)
    slot_row = idx3 * BC + bidx * C + jnp.minimum(pos_i, C - 1)
    drop_row = EBC + bidx * S + sidx
    cmb_ref[:] = jnp.where(within, slot_row, drop_row)
    init_ref[:] = x * mp3


def _router(x, Wr, br):
    return pl.pallas_call(
        _router_body,
        out_shape=[
            jax.ShapeDtypeStruct((E, BC), jnp.int32),
            jax.ShapeDtypeStruct((B, S), jnp.int32),
            jax.ShapeDtypeStruct((E, 1, BC), jnp.float32),
            jax.ShapeDtypeStruct((B, S, D), jnp.float32),
        ],
    )(x, Wr, br.reshape(1, E))


def _mlp_body(buf_ref, scs_ref, w1_ref, b1_ref, w2_ref, b2_ref, init_ref,
              y_ref):
    i = pl.program_id(0)

    @pl.when(i < E)
    def _():
        h = jnp.dot(buf_ref[:], w1_ref[0],
                    preferred_element_type=jnp.float32) + b1_ref[0]
        h = 0.5 * h * (1.0 + jnp.tanh(SQ2PI * (h + 0.044715 * (h * h * h))))
        y = jnp.dot(h, w2_ref[0],
                    preferred_element_type=jnp.float32) + b2_ref[0]
        y_ref[:] = y * scs_ref[0, 0, :][:, None]

    @pl.when(i >= E)
    def _():
        y_ref[:] = init_ref[:]


def _mlp(buf, scs, W1, b1, W2, b2, init2):
    ee = lambda i: jnp.minimum(i, E - 1)
    return pl.pallas_call(
        _mlp_body,
        grid=(E + BS // BC,),
        in_specs=[
            pl.BlockSpec((BC, D), lambda i: (ee(i), 0)),
            pl.BlockSpec((1, 1, BC), lambda i: (ee(i), 0, 0)),
            pl.BlockSpec((1, D, F), lambda i: (ee(i), 0, 0)),
            pl.BlockSpec((1, 1, F), lambda i: (ee(i), 0, 0)),
            pl.BlockSpec((1, F, D), lambda i: (ee(i), 0, 0)),
            pl.BlockSpec((1, 1, D), lambda i: (ee(i), 0, 0)),
            pl.BlockSpec((BC, D), lambda i: (jnp.maximum(i - E, 0), 0)),
        ],
        out_specs=pl.BlockSpec((BC, D), lambda i: (i, 0)),
        out_shape=jax.ShapeDtypeStruct((NROWS, D), jnp.float32),
        compiler_params=pltpu.CompilerParams(
            dimension_semantics=("parallel",)),
    )(buf, scs, W1, b1.reshape(E, 1, F), W2, b2.reshape(E, 1, D), init2)


def _sc_gather(table, idx, n_out):
    nw = 32
    b_per_w = n_out // nw
    ch = 64
    nch = b_per_w // ch
    mesh = plsc.VectorSubcoreMesh(core_axis_name="c", subcore_axis_name="s")

    @functools.partial(
        pl.kernel, mesh=mesh,
        out_type=jax.ShapeDtypeStruct((n_out, D), jnp.float32),
        scratch_types=[
            pltpu.VMEM((ch,), jnp.int32),
            pltpu.VMEM((ch, D), jnp.float32),
            pltpu.SemaphoreType.DMA,
        ],
    )
    def k(table_hbm, idx_hbm, out_hbm, idx_v, rows_v, sem):
        wid = jax.lax.axis_index("s") * 2 + jax.lax.axis_index("c")
        base = wid * b_per_w

        @pl.loop(0, nch)
        def _(j):
            off = base + j * ch
            pltpu.sync_copy(idx_hbm.at[pl.ds(off, ch)], idx_v)
            pltpu.async_copy(table_hbm.at[idx_v], rows_v, sem).wait()
            pltpu.sync_copy(rows_v, out_hbm.at[pl.ds(off, ch)])

    return k(table, idx)


def kernel(hidden_states, Wr, br, W1, b1, W2, b2):
    dsp, cmb, scs, init = _router(hidden_states, Wr, br)
    x2 = hidden_states.reshape(BS, D)
    buf = _sc_gather(x2, dsp.reshape(EBC), EBC)
    ybig = _mlp(buf, scs, W1, b1, W2, b2, init.reshape(BS, D))
    out = _sc_gather(ybig, cmb.reshape(BS), BS)
    return out.reshape(B, S, D)


# speedup vs baseline: 1.0618x; 1.0618x over previous
import functools

import jax
import jax.numpy as jnp
import numpy as np
from jax.experimental import pallas as pl
from jax.experimental.pallas import tpu as pltpu
from jax.experimental.pallas import tpu_sc as plsc

B, S, D = 2, 2048, 768
E, C, F = 64, 64, 3072
BS = B * S
BC = B * C
EBC = E * BC
NROWS = EBC + BS
SQ2PI = 0.7978845608028654

_HI = jax.lax.Precision.HIGHEST


def _fused_body(x_ref, wr_ref, br_ref, w1_ref, b1_ref, w2_ref, b2_ref,
                y_ref, cmb_ref, dsp_v, dsp_s, scs_v, mp_v, xb_s, sem):
    i = pl.program_id(0)

    @pl.when(i == 0)
    def _router():
        logits = jnp.dot(x_ref[:], wr_ref[:],
                         preferred_element_type=jnp.float32) + br_ref[:]
        l3 = logits.reshape(B, S, E)
        m3 = jnp.max(l3, axis=-1, keepdims=True)
        ssum = jnp.sum(jnp.exp(l3 - m3), axis=-1, keepdims=True)
        mp3 = 1.0 / ssum
        ie = jax.lax.broadcasted_iota(jnp.int32, (B, S, E), 2)
        idx3 = jnp.min(jnp.where(l3 == m3, ie, E), axis=-1)
        oh = (ie == idx3[:, :, None]).astype(jnp.float32)
        cum = oh
        k = 1
        while k < S:
            cum = cum + jnp.concatenate(
                [jnp.zeros((B, k, E), jnp.float32), cum[:, :S - k, :]],
                axis=1)
            k *= 2
        posf = jnp.sum(cum * oh, axis=-1) - 1.0
        ic = jax.lax.broadcasted_iota(jnp.int32, (B, S, C), 2).astype(
            jnp.float32)
        poh = (ic == posf[:, :, None]).astype(jnp.float32)
        s1 = jax.lax.broadcasted_iota(jnp.int32, (B, S, E), 1).astype(
            jnp.float32) + 1.0
        dn = (((0,), (0,)), ((), ()))
        dsp_cols, sc_cols = [], []
        for b in range(B):
            stb = jax.lax.dot_general(oh[b] * s1[b], poh[b], dn,
                                      precision=_HI)
            scb = jax.lax.dot_general(oh[b] * mp3[b], poh[b], dn,
                                      precision=_HI)
            t = stb.astype(jnp.int32) - 1
            dsp_cols.append(jnp.maximum(t, 0) + b * S)
            sc_cols.append(scb)
        dsp_v[:] = jnp.concatenate(dsp_cols, axis=1)
        scs_v[:] = jnp.concatenate(sc_cols, axis=1)
        mp_v[:] = mp3.reshape(BS, 1)
        pos_i = posf.astype(jnp.int32)
        within = posf < float(C)
        bidx = jax.lax.broadcasted_iota(jnp.int32, (B, S), 0)
        sidx = jax.lax.broadcasted_iota(jnp.int32, (B, S), 1)
        slot_row = idx3 * BC + bidx * C + jnp.minimum(pos_i, C - 1)
        drop_row = EBC + bidx * S + sidx
        cmb_ref[:] = jnp.where(within, slot_row, drop_row)
        pltpu.make_async_copy(dsp_v, dsp_s, sem).start()
        pltpu.make_async_copy(dsp_v, dsp_s, sem).wait()

    @pl.when(i < E)
    def _expert():
        def gather(r, carry):
            t = dsp_s[i, r]
            xb_s[pl.ds(r, 1), :] = x_ref[pl.ds(t, 1), :]
            return carry

        jax.lax.fori_loop(0, BC, gather, 0, unroll=True)
        h = jnp.dot(xb_s[:], w1_ref[0],
                    preferred_element_type=jnp.float32) + b1_ref[0]
        h = 0.5 * h * (1.0 + jnp.tanh(SQ2PI * (h + 0.044715 * (h * h * h))))
        y = jnp.dot(h, w2_ref[0],
                    preferred_element_type=jnp.float32) + b2_ref[0]
        s = scs_v[pl.ds(i, 1), :].reshape(BC)
        y_ref[:] = y * s[:, None]

    @pl.when(i >= E)
    def _init():
        base = (i - E) * BC
        y_ref[:] = x_ref[pl.ds(base, BC), :] * mp_v[pl.ds(base, BC), :]


def _fused(x2, Wr, br, W1, b1, W2, b2):
    ee = lambda i: jnp.minimum(i, E - 1)
    return pl.pallas_call(
        _fused_body,
        grid=(E + BS // BC,),
        in_specs=[
            pl.BlockSpec((BS, D), lambda i: (0, 0)),
            pl.BlockSpec((D, E), lambda i: (0, 0)),
            pl.BlockSpec((1, E), lambda i: (0, 0)),
            pl.BlockSpec((1, D, F), lambda i: (ee(i), 0, 0)),
            pl.BlockSpec((1, 1, F), lambda i: (ee(i), 0, 0)),
            pl.BlockSpec((1, F, D), lambda i: (ee(i), 0, 0)),
            pl.BlockSpec((1, 1, D), lambda i: (ee(i), 0, 0)),
        ],
        out_specs=[
            pl.BlockSpec((BC, D), lambda i: (i, 0)),
            pl.BlockSpec((B, S), lambda i: (0, 0)),
        ],
        out_shape=[
            jax.ShapeDtypeStruct((NROWS, D), jnp.float32),
            jax.ShapeDtypeStruct((B, S), jnp.int32),
        ],
        scratch_shapes=[
            pltpu.VMEM((E, BC), jnp.int32),
            pltpu.SMEM((E, BC), jnp.int32),
            pltpu.VMEM((E, BC), jnp.float32),
            pltpu.VMEM((BS, 1), jnp.float32),
            pltpu.VMEM((BC, D), jnp.float32),
            pltpu.SemaphoreType.DMA,
        ],
        compiler_params=pltpu.CompilerParams(
            dimension_semantics=("arbitrary",),
            vmem_limit_bytes=64 * 1024 * 1024),
    )(x2, Wr, br.reshape(1, E), W1, b1.reshape(E, 1, F), W2,
      b2.reshape(E, 1, D))


def _sc_gather(table, idx, n_out):
    nw = 32
    b_per_w = n_out // nw
    ch = 64
    nch = b_per_w // ch
    mesh = plsc.VectorSubcoreMesh(core_axis_name="c", subcore_axis_name="s")

    @functools.partial(
        pl.kernel, mesh=mesh,
        out_type=jax.ShapeDtypeStruct((n_out, D), jnp.float32),
        scratch_types=[
            pltpu.VMEM((ch,), jnp.int32),
            pltpu.VMEM((ch, D), jnp.float32),
            pltpu.SemaphoreType.DMA,
        ],
    )
    def k(table_hbm, idx_hbm, out_hbm, idx_v, rows_v, sem):
        wid = jax.lax.axis_index("s") * 2 + jax.lax.axis_index("c")
        base = wid * b_per_w

        @pl.loop(0, nch)
        def _(j):
            off = base + j * ch
            pltpu.sync_copy(idx_hbm.at[pl.ds(off, ch)], idx_v)
            pltpu.async_copy(table_hbm.at[idx_v], rows_v, sem).wait()
            pltpu.sync_copy(rows_v, out_hbm.at[pl.ds(off, ch)])

    return k(table, idx)


def kernel(hidden_states, Wr, br, W1, b1, W2, b2):
    x2 = hidden_states.reshape(BS, D)
    ybig, cmb = _fused(x2, Wr, br, W1, b1, W2, b2)
    out = _sc_gather(ybig, cmb.reshape(BS), BS)
    return out.reshape(B, S, D)


# speedup vs baseline: 1.0723x; 1.0099x over previous
import jax
import jax.numpy as jnp
import numpy as np
from jax.experimental import pallas as pl
from jax.experimental.pallas import tpu as pltpu

B, S, D = 2, 2048, 768
E, C, F = 64, 64, 3072
BS = B * S
BC = B * C
NF = 2
FT = F // NF
NI = BS // BC
SQ2PI = 0.7978845608028654

_HI = jax.lax.Precision.HIGHEST


def _fused_body(x_ref, wr_ref, br_ref, w1_ref, b1_ref, w2_ref, b2_ref,
                out_ref, dsp_v, dsp_s, mp_v, xb_s, acc_s, sem):
    i = pl.program_id(0)
    f = pl.program_id(1)

    @pl.when((i < NI) & (f == 0))
    def _init():
        base = i * BC
        out_ref[pl.ds(base, BC), :] = x_ref[pl.ds(base, BC), :]

    @pl.when((i == NI) & (f == 0))
    def _router():
        logits = jnp.dot(x_ref[:], wr_ref[:],
                         preferred_element_type=jnp.float32) + br_ref[:]
        l3 = logits.reshape(B, S, E)
        m3 = jnp.max(l3, axis=-1, keepdims=True)
        ssum = jnp.sum(jnp.exp(l3 - m3), axis=-1, keepdims=True)
        mp3 = 1.0 / ssum
        ie = jax.lax.broadcasted_iota(jnp.int32, (B, S, E), 2)
        idx3 = jnp.min(jnp.where(l3 == m3, ie, E), axis=-1)
        oh = (ie == idx3[:, :, None]).astype(jnp.float32)
        cum = oh
        k = 1
        while k < S:
            cum = cum + jnp.concatenate(
                [jnp.zeros((B, k, E), jnp.float32), cum[:, :S - k, :]],
                axis=1)
            k *= 2
        posf = jnp.sum(cum * oh, axis=-1) - 1.0
        ic = jax.lax.broadcasted_iota(jnp.int32, (B, S, C), 2).astype(
            jnp.float32)
        poh = (ic == posf[:, :, None]).astype(jnp.float32)
        s1 = jax.lax.broadcasted_iota(jnp.int32, (B, S, E), 1).astype(
            jnp.float32) + 1.0
        dn = (((0,), (0,)), ((), ()))
        dsp_cols = []
        for b in range(B):
            stb = jax.lax.dot_general(oh[b] * s1[b], poh[b], dn,
                                      precision=_HI)
            t = stb.astype(jnp.int32) - 1
            dsp_cols.append(jnp.where(t >= 0, t + b * S, -1))
        dsp_v[:] = jnp.concatenate(dsp_cols, axis=1)
        mp_v[:] = mp3.reshape(BS, 1)
        pltpu.make_async_copy(dsp_v, dsp_s, sem).start()
        pltpu.make_async_copy(dsp_v, dsp_s, sem).wait()

    @pl.when(i > NI)
    def _expert():
        e = i - NI - 1

        @pl.when(f == 0)
        def _gather():
            def gather(r, carry):
                t = jnp.maximum(dsp_s[e, r], 0)
                xb_s[pl.ds(r, 1), :] = x_ref[pl.ds(t, 1), :]
                return carry

            jax.lax.fori_loop(0, BC, gather, 0, unroll=True)

        h = jnp.dot(xb_s[:], w1_ref[0],
                    preferred_element_type=jnp.float32) + b1_ref[0]
        h = 0.5 * h * (1.0 + jnp.tanh(SQ2PI * (h + 0.044715 * (h * h * h))))
        part = jnp.dot(h, w2_ref[0], preferred_element_type=jnp.float32)

        @pl.when(f == 0)
        def _acc0():
            acc_s[:] = part + b2_ref[0]

        @pl.when(f == NF - 1)
        def _scatter():
            acc_s[:] += part

            def scatter(r, carry):
                t = dsp_s[e, r]

                @pl.when(t >= 0)
                def _():
                    out_ref[pl.ds(t, 1), :] = acc_s[pl.ds(r, 1), :]

                return carry

            jax.lax.fori_loop(0, BC, scatter, 0, unroll=True)

    @pl.when((i == NI + E) & (f == NF - 1))
    def _scale():
        out_ref[:] = out_ref[:] * mp_v[:]


def kernel(hidden_states, Wr, br, W1, b1, W2, b2):
    x2 = hidden_states.reshape(BS, D)
    ei = lambda i: jnp.maximum(i - NI - 1, 0)
    fw = lambda i, f: jnp.where(i <= NI, 0, f)
    out = pl.pallas_call(
        _fused_body,
        grid=(NI + 1 + E, NF),
        in_specs=[
            pl.BlockSpec((BS, D), lambda i, f: (0, 0)),
            pl.BlockSpec((D, E), lambda i, f: (0, 0)),
            pl.BlockSpec((1, E), lambda i, f: (0, 0)),
            pl.BlockSpec((1, D, FT), lambda i, f: (ei(i), 0, fw(i, f))),
            pl.BlockSpec((1, 1, FT), lambda i, f: (ei(i), 0, fw(i, f))),
            pl.BlockSpec((1, FT, D), lambda i, f: (ei(i), fw(i, f), 0)),
            pl.BlockSpec((1, 1, D), lambda i, f: (ei(i), 0, 0)),
        ],
        out_specs=pl.BlockSpec((BS, D), lambda i, f: (0, 0)),
        out_shape=jax.ShapeDtypeStruct((BS, D), jnp.float32),
        scratch_shapes=[
            pltpu.VMEM((E, BC), jnp.int32),
            pltpu.SMEM((E, BC), jnp.int32),
            pltpu.VMEM((BS, 1), jnp.float32),
            pltpu.VMEM((BC, D), jnp.float32),
            pltpu.VMEM((BC, D), jnp.float32),
            pltpu.SemaphoreType.DMA,
        ],
        compiler_params=pltpu.CompilerParams(
            dimension_semantics=("arbitrary", "arbitrary"),
            vmem_limit_bytes=64 * 1024 * 1024),
    )(x2, Wr, br.reshape(1, E), W1, b1.reshape(E, 1, F), W2,
      b2.reshape(E, 1, D))
    return out.reshape(B, S, D)


# speedup vs baseline: 1.0832x; 1.0101x over previous
import functools

import jax
import jax.numpy as jnp
import numpy as np
from jax.experimental import pallas as pl
from jax.experimental.pallas import tpu as pltpu
from jax.experimental.pallas import tpu_sc as plsc

B, S, D = 2, 2048, 768
E, C, F = 64, 64, 3072
BS = B * S
BC = B * C
EBC = E * BC
NROWS = EBC + BS
SQ2PI = 0.7978845608028654

_HI = jax.lax.Precision.HIGHEST


def _fused_body(x_ref, wr_ref, br_ref, w1_ref, b1_ref, w2_ref, b2_ref,
                y_ref, cmb_ref, dsp_v, dsp_s, scs_v, mp_v, xb_s, sem):
    i = pl.program_id(0)

    @pl.when(i == 0)
    def _router():
        logits = jnp.dot(x_ref[:], wr_ref[:],
                         preferred_element_type=jnp.float32) + br_ref[:]
        l3 = logits.reshape(B, S, E)
        m3 = jnp.max(l3, axis=-1, keepdims=True)
        ssum = jnp.sum(jnp.exp(l3 - m3), axis=-1, keepdims=True)
        mp3 = 1.0 / ssum
        ie = jax.lax.broadcasted_iota(jnp.int32, (B, S, E), 2)
        idx3 = jnp.min(jnp.where(l3 == m3, ie, E), axis=-1)
        oh = (ie == idx3[:, :, None]).astype(jnp.float32)
        cum = oh
        k = 1
        while k < S:
            cum = cum + jnp.concatenate(
                [jnp.zeros((B, k, E), jnp.float32), cum[:, :S - k, :]],
                axis=1)
            k *= 2
        posf = jnp.sum(cum * oh, axis=-1) - 1.0
        ic = jax.lax.broadcasted_iota(jnp.int32, (B, S, C), 2).astype(
            jnp.float32)
        poh = (ic == posf[:, :, None]).astype(jnp.float32)
        s1 = jax.lax.broadcasted_iota(jnp.int32, (B, S, E), 1).astype(
            jnp.float32) + 1.0
        dn = (((0,), (0,)), ((), ()))
        dsp_cols, sc_cols = [], []
        for b in range(B):
            stb = jax.lax.dot_general(oh[b] * s1[b], poh[b], dn,
                                      precision=_HI)
            scb = jax.lax.dot_general(oh[b] * mp3[b], poh[b], dn,
                                      precision=_HI)
            t = stb.astype(jnp.int32) - 1
            dsp_cols.append(jnp.maximum(t, 0) + b * S)
            sc_cols.append(scb)
        dsp_v[:] = jnp.concatenate(dsp_cols, axis=1)
        scs_v[:] = jnp.concatenate(sc_cols, axis=1)
        mp_v[:] = mp3.reshape(BS, 1)
        pos_i = posf.astype(jnp.int32)
        within = posf < float(C)
        bidx = jax.lax.broadcasted_iota(jnp.int32, (B, S), 0)
        sidx = jax.lax.broadcasted_iota(jnp.int32, (B, S), 1)
        slot_row = idx3 * BC + bidx * C + jnp.minimum(pos_i, C - 1)
        drop_row = EBC + bidx * S + sidx
        cmb_ref[:] = jnp.where(within, slot_row, drop_row)
        pltpu.make_async_copy(dsp_v, dsp_s, sem).start()
        pltpu.make_async_copy(dsp_v, dsp_s, sem).wait()

    @pl.when(i < E)
    def _expert():
        def gather(r, carry):
            t = dsp_s[i, r]
            xb_s[pl.ds(r, 1), :] = x_ref[pl.ds(t, 1), :]
            return carry

        jax.lax.fori_loop(0, BC, gather, 0, unroll=True)
        h = jnp.dot(xb_s[:], w1_ref[0],
                    preferred_element_type=jnp.float32) + b1_ref[0]
        h = 0.5 * h * (1.0 + jnp.tanh(SQ2PI * (h + 0.044715 * (h * h * h))))
        y = jnp.dot(h, w2_ref[0],
                    preferred_element_type=jnp.float32) + b2_ref[0]
        s = scs_v[pl.ds(i, 1), :].reshape(BC)
        y_ref[:] = y * s[:, None]

    @pl.when(i >= E)
    def _init():
        base = (i - E) * BC
        y_ref[:] = x_ref[pl.ds(base, BC), :] * mp_v[pl.ds(base, BC), :]


def _fused(x2, Wr, br, W1, b1, W2, b2):
    ee = lambda i: jnp.minimum(i, E - 1)
    return pl.pallas_call(
        _fused_body,
        grid=(E + BS // BC,),
        in_specs=[
            pl.BlockSpec((BS, D), lambda i: (0, 0)),
            pl.BlockSpec((D, E), lambda i: (0, 0)),
            pl.BlockSpec((1, E), lambda i: (0, 0)),
            pl.BlockSpec((1, D, F), lambda i: (ee(i), 0, 0)),
            pl.BlockSpec((1, 1, F), lambda i: (ee(i), 0, 0)),
            pl.BlockSpec((1, F, D), lambda i: (ee(i), 0, 0)),
            pl.BlockSpec((1, 1, D), lambda i: (ee(i), 0, 0)),
        ],
        out_specs=[
            pl.BlockSpec((BC, D), lambda i: (i, 0)),
            pl.BlockSpec((B, S), lambda i: (0, 0)),
        ],
        out_shape=[
            jax.ShapeDtypeStruct((NROWS, D), jnp.float32),
            jax.ShapeDtypeStruct((B, S), jnp.int32),
        ],
        scratch_shapes=[
            pltpu.VMEM((E, BC), jnp.int32),
            pltpu.SMEM((E, BC), jnp.int32),
            pltpu.VMEM((E, BC), jnp.float32),
            pltpu.VMEM((BS, 1), jnp.float32),
            pltpu.VMEM((BC, D), jnp.float32),
            pltpu.SemaphoreType.DMA,
        ],
        compiler_params=pltpu.CompilerParams(
            dimension_semantics=("arbitrary",),
            vmem_limit_bytes=64 * 1024 * 1024),
    )(x2, Wr, br.reshape(1, E), W1, b1.reshape(E, 1, F), W2,
      b2.reshape(E, 1, D))


def _sc_gather(table, idx, n_out):
    nw = 32
    b_per_w = n_out // nw
    ch = 128
    nch = b_per_w // ch
    mesh = plsc.VectorSubcoreMesh(core_axis_name="c", subcore_axis_name="s")

    @functools.partial(
        pl.kernel, mesh=mesh,
        out_type=jax.ShapeDtypeStruct((n_out, D), jnp.float32),
        scratch_types=[
            pltpu.VMEM((ch,), jnp.int32),
            pltpu.VMEM((ch, D), jnp.float32),
            pltpu.SemaphoreType.DMA,
        ],
    )
    def k(table_hbm, idx_hbm, out_hbm, idx_v, rows_v, sem):
        wid = jax.lax.axis_index("s") * 2 + jax.lax.axis_index("c")
        base = wid * b_per_w

        @pl.loop(0, nch)
        def _(j):
            off = base + j * ch
            pltpu.sync_copy(idx_hbm.at[pl.ds(off, ch)], idx_v)
            pltpu.async_copy(table_hbm.at[idx_v], rows_v, sem).wait()
            pltpu.sync_copy(rows_v, out_hbm.at[pl.ds(off, ch)])

    return k(table, idx)


def kernel(hidden_states, Wr, br, W1, b1, W2, b2):
    x2 = hidden_states.reshape(BS, D)
    ybig, cmb = _fused(x2, Wr, br, W1, b1, W2, b2)
    out = _sc_gather(ybig, cmb.reshape(BS), BS)
    return out.reshape(B, S, D)
